# baseline (speedup 1.0000x reference)
import functools

import jax
import jax.numpy as jnp
from jax import lax
from jax.experimental import pallas as pl
from jax.experimental.pallas import tpu as pltpu
from jax.experimental.pallas import tpu_sc as plsc

_HIDDEN = 32
_BATCH = 1024
_MARGIN = 1.0
_LAMB = 0.01
_REG = 0.01

_NC = 2
_NS = 16
_NW = _NC * _NS
_L = 16
_REL_BASE = 10000
_TBL = _REL_BASE * 2
_PPC = _BATCH // _NC
_PPW = _PPC // _NS
_G = _PPC // _L


def _tec_body(tbl_hbm, ih_hbm, ir_hbm, it_hbm, out_hbm,
              row0_v, row1_v, ihp, irp, itp, ihn, irn, itn,
              pp_v, nn_v, np_v, fin_v, acc_v, shared, sem, sem1):
    sid = lax.axis_index("c")
    t = lax.axis_index("s")
    wid = t * _NC + sid
    p0 = sid * _PPC

    c_row1 = pltpu.async_copy(tbl_hbm.at[t + _NS], row1_v, sem1)
    cps = [
        pltpu.async_copy(tbl_hbm.at[t], row0_v, sem),
        pltpu.async_copy(ih_hbm.at[pl.ds(p0, _PPC)], ihp, sem),
        pltpu.async_copy(ir_hbm.at[pl.ds(p0, _PPC)], irp, sem),
        pltpu.async_copy(it_hbm.at[pl.ds(p0, _PPC)], itp, sem),
        pltpu.async_copy(ih_hbm.at[pl.ds(p0 + _BATCH, _PPC)], ihn, sem),
        pltpu.async_copy(ir_hbm.at[pl.ds(p0 + _BATCH, _PPC)], irn, sem),
        pltpu.async_copy(it_hbm.at[pl.ds(p0 + _BATCH, _PPC)], itn, sem),
    ]
    for c in cps:
        c.wait()

    zero = jnp.zeros((_L,), jnp.float32)
    h_acc, r_acc, t_acc = zero, zero, zero

    for pi, row in enumerate((row0_v, row1_v)):
        if pi == 1:
            c_row1.wait()
        for g in range(_G):
            s = g * _L
            vih_p = ihp[pl.ds(s, _L)]
            vir_p = irp[pl.ds(s, _L)]
            vit_p = itp[pl.ds(s, _L)]
            vih_n = ihn[pl.ds(s, _L)]
            vir_n = irn[pl.ds(s, _L)]
            vit_n = itn[pl.ds(s, _L)]
            vhp = plsc.load_gather(row, [vih_p])
            vrp = plsc.load_gather(row, [vir_p])
            vtp = plsc.load_gather(row, [vit_p])
            vhn = plsc.load_gather(row, [vih_n])
            vrn = plsc.load_gather(row, [vir_n])
            vtn = plsc.load_gather(row, [vit_n])
            ep = jnp.abs(vhp + vrp - vtp)
            en = jnp.abs(vhn + vrn - vtn)
            spp = ep * ep
            snn = en * en
            snp = ep * en
            if pi == 1:
                spp = spp + pp_v[pl.ds(s, _L)]
                snn = snn + nn_v[pl.ds(s, _L)]
                snp = snp + np_v[pl.ds(s, _L)]
            h_acc = h_acc + vhp * vhp + vhn * vhn
            r_acc = r_acc + vrp * vrp + vrn * vrn
            t_acc = t_acc + vtp * vtp + vtn * vtn
            pp_v[pl.ds(s, _L)] = spp
            nn_v[pl.ds(s, _L)] = snn
            np_v[pl.ds(s, _L)] = snp

    pltpu.sync_copy(pp_v, shared.at[0, t])
    pltpu.sync_copy(nn_v, shared.at[1, t])
    pltpu.sync_copy(np_v, shared.at[2, t])
    plsc.subcore_barrier()
    pltpu.sync_copy(shared.at[:, :, pl.ds(t * _PPW, _PPW)], fin_v)

    m_acc, w_acc = zero, zero
    for half in range(_PPW // _L):
        cpp, cnn, cnp = zero, zero, zero
        for w in range(_NS):
            cpp = cpp + fin_v[0, w, pl.ds(half * _L, _L)]
            cnn = cnn + fin_v[1, w, pl.ds(half * _L, _L)]
            cnp = cnp + fin_v[2, w, pl.ds(half * _L, _L)]
        m = 2.0 * cnp * cnp - cpp * cpp - cnn * cnn + _MARGIN
        m_acc = m_acc + jnp.maximum(m, 0.0)
        w_acc = w_acc + (_MARGIN - m)

    acc_v[0, :] = m_acc
    acc_v[1, :] = w_acc
    acc_v[2, :] = h_acc
    acc_v[3, :] = r_acc
    acc_v[4, :] = t_acc
    pltpu.sync_copy(acc_v, out_hbm.at[wid])


_sc_call = functools.partial(
    pl.kernel,
    mesh=plsc.VectorSubcoreMesh(core_axis_name="c", subcore_axis_name="s"),
    out_type=jax.ShapeDtypeStruct((_NW, 5, _L), jnp.float32),
    compiler_params=pltpu.CompilerParams(
        needs_layout_passes=False, use_tc_tiling_on_sc=False),
    scratch_types=[
        pltpu.VMEM((_TBL,), jnp.float32),
        pltpu.VMEM((_TBL,), jnp.float32),
        pltpu.VMEM((_PPC,), jnp.int32),
        pltpu.VMEM((_PPC,), jnp.int32),
        pltpu.VMEM((_PPC,), jnp.int32),
        pltpu.VMEM((_PPC,), jnp.int32),
        pltpu.VMEM((_PPC,), jnp.int32),
        pltpu.VMEM((_PPC,), jnp.int32),
        pltpu.VMEM((_PPC,), jnp.float32),
        pltpu.VMEM((_PPC,), jnp.float32),
        pltpu.VMEM((_PPC,), jnp.float32),
        pltpu.VMEM((3, _NS, _PPW), jnp.float32),
        pltpu.VMEM((5, _L), jnp.float32),
        pltpu.VMEM_SHARED((3, _NS, _PPC), jnp.float32),
        pltpu.SemaphoreType.DMA,
        pltpu.SemaphoreType.DMA,
    ],
)(_tec_body)


def kernel(input, ent_embeddings, rel_embeddings):
    ih = input[:, 0]
    ir = input[:, 1] + _REL_BASE
    it = input[:, 2]
    tbl = jnp.concatenate(
        [ent_embeddings[:_REL_BASE], rel_embeddings], axis=0).T
    parts = _sc_call(tbl, ih, ir, it)
    s_margin = jnp.sum(parts[:, 0, :])
    s_wr = jnp.maximum(jnp.sum(parts[:, 1, :]), 0.0)
    s_h = jnp.sum(parts[:, 2, :])
    s_r = jnp.sum(parts[:, 3, :])
    s_t = jnp.sum(parts[:, 4, :])
    return (s_margin / _BATCH
            + _LAMB * jnp.sqrt(s_wr)
            + _REG * (jnp.sqrt(s_h) + jnp.sqrt(s_r) + jnp.sqrt(s_t)))


# speedup vs baseline: 1.0170x; 1.0170x over previous
import functools

import jax
import jax.numpy as jnp
from jax import lax
from jax.experimental import pallas as pl
from jax.experimental.pallas import tpu as pltpu
from jax.experimental.pallas import tpu_sc as plsc

_HIDDEN = 32
_BATCH = 1024
_MARGIN = 1.0
_LAMB = 0.01
_REG = 0.01

_NC = 2
_NS = 16
_NW = _NC * _NS
_L = 16
_REL_BASE = 10000
_TBL = _REL_BASE * 2
_PPC = _BATCH // _NC
_PPW = _PPC // _NS
_G = _PPC // _L


def _tec_body(tbl_hbm, ih_hbm, ir_hbm, it_hbm, out_hbm,
              row0_v, row1_v, ihp, irp, itp, ihn, irn, itn,
              pp_v, nn_v, np_v, fin_v, acc_v, shared, sem):
    sid = lax.axis_index("c")
    t = lax.axis_index("s")
    wid = t * _NC + sid
    p0 = sid * _PPC

    cps = [
        pltpu.async_copy(tbl_hbm.at[t], row0_v, sem),
        pltpu.async_copy(tbl_hbm.at[t + _NS], row1_v, sem),
        pltpu.async_copy(ih_hbm.at[pl.ds(p0, _PPC)], ihp, sem),
        pltpu.async_copy(ir_hbm.at[pl.ds(p0, _PPC)], irp, sem),
        pltpu.async_copy(it_hbm.at[pl.ds(p0, _PPC)], itp, sem),
        pltpu.async_copy(ih_hbm.at[pl.ds(p0 + _BATCH, _PPC)], ihn, sem),
        pltpu.async_copy(ir_hbm.at[pl.ds(p0 + _BATCH, _PPC)], irn, sem),
        pltpu.async_copy(it_hbm.at[pl.ds(p0 + _BATCH, _PPC)], itn, sem),
    ]
    for c in cps:
        c.wait()

    zero = jnp.zeros((_L,), jnp.float32)
    h_acc, r_acc, t_acc = zero, zero, zero

    for g in range(_G):
        s = g * _L
        vih_p = ihp[pl.ds(s, _L)]
        vir_p = irp[pl.ds(s, _L)]
        vit_p = itp[pl.ds(s, _L)]
        vih_n = ihn[pl.ds(s, _L)]
        vir_n = irn[pl.ds(s, _L)]
        vit_n = itn[pl.ds(s, _L)]
        spp, snn, snp = zero, zero, zero
        for row in (row0_v, row1_v):
            vhp = plsc.load_gather(row, [vih_p])
            vrp = plsc.load_gather(row, [vir_p])
            vtp = plsc.load_gather(row, [vit_p])
            vhn = plsc.load_gather(row, [vih_n])
            vrn = plsc.load_gather(row, [vir_n])
            vtn = plsc.load_gather(row, [vit_n])
            ep = jnp.abs(vhp + vrp - vtp)
            en = jnp.abs(vhn + vrn - vtn)
            spp = spp + ep * ep
            snn = snn + en * en
            snp = snp + ep * en
            h_acc = h_acc + vhp * vhp + vhn * vhn
            r_acc = r_acc + vrp * vrp + vrn * vrn
            t_acc = t_acc + vtp * vtp + vtn * vtn
        pp_v[pl.ds(s, _L)] = spp
        nn_v[pl.ds(s, _L)] = snn
        np_v[pl.ds(s, _L)] = snp

    pltpu.sync_copy(pp_v, shared.at[0, t])
    pltpu.sync_copy(nn_v, shared.at[1, t])
    pltpu.sync_copy(np_v, shared.at[2, t])
    plsc.subcore_barrier()
    pltpu.sync_copy(shared.at[:, :, pl.ds(t * _PPW, _PPW)], fin_v)

    m_acc, w_acc = zero, zero
    for half in range(_PPW // _L):
        cpp, cnn, cnp = zero, zero, zero
        for w in range(_NS):
            cpp = cpp + fin_v[0, w, pl.ds(half * _L, _L)]
            cnn = cnn + fin_v[1, w, pl.ds(half * _L, _L)]
            cnp = cnp + fin_v[2, w, pl.ds(half * _L, _L)]
        m = 2.0 * cnp * cnp - cpp * cpp - cnn * cnn + _MARGIN
        m_acc = m_acc + jnp.maximum(m, 0.0)
        w_acc = w_acc + (_MARGIN - m)

    acc_v[0, :] = m_acc
    acc_v[1, :] = w_acc
    acc_v[2, :] = h_acc
    acc_v[3, :] = r_acc
    acc_v[4, :] = t_acc
    pltpu.sync_copy(acc_v, out_hbm.at[wid])


_sc_call = functools.partial(
    pl.kernel,
    mesh=plsc.VectorSubcoreMesh(core_axis_name="c", subcore_axis_name="s"),
    out_type=jax.ShapeDtypeStruct((_NW, 5, _L), jnp.float32),
    compiler_params=pltpu.CompilerParams(
        needs_layout_passes=False, use_tc_tiling_on_sc=False),
    scratch_types=[
        pltpu.VMEM((_TBL,), jnp.float32),
        pltpu.VMEM((_TBL,), jnp.float32),
        pltpu.VMEM((_PPC,), jnp.int32),
        pltpu.VMEM((_PPC,), jnp.int32),
        pltpu.VMEM((_PPC,), jnp.int32),
        pltpu.VMEM((_PPC,), jnp.int32),
        pltpu.VMEM((_PPC,), jnp.int32),
        pltpu.VMEM((_PPC,), jnp.int32),
        pltpu.VMEM((_PPC,), jnp.float32),
        pltpu.VMEM((_PPC,), jnp.float32),
        pltpu.VMEM((_PPC,), jnp.float32),
        pltpu.VMEM((3, _NS, _PPW), jnp.float32),
        pltpu.VMEM((5, _L), jnp.float32),
        pltpu.VMEM_SHARED((3, _NS, _PPC), jnp.float32),
        pltpu.SemaphoreType.DMA,
    ],
)(_tec_body)


def kernel(input, ent_embeddings, rel_embeddings):
    ih = input[:, 0]
    ir = input[:, 1] + _REL_BASE
    it = input[:, 2]
    tbl = jnp.concatenate(
        [ent_embeddings[:_REL_BASE], rel_embeddings], axis=0).T
    parts = _sc_call(tbl, ih, ir, it)
    s_margin = jnp.sum(parts[:, 0, :])
    s_wr = jnp.maximum(jnp.sum(parts[:, 1, :]), 0.0)
    s_h = jnp.sum(parts[:, 2, :])
    s_r = jnp.sum(parts[:, 3, :])
    s_t = jnp.sum(parts[:, 4, :])
    return (s_margin / _BATCH
            + _LAMB * jnp.sqrt(s_wr)
            + _REG * (jnp.sqrt(s_h) + jnp.sqrt(s_r) + jnp.sqrt(s_t)))
